# baseline (speedup 1.0000x reference)
import functools
import math

import numpy as np

import jax
import jax.numpy as jnp
from jax import lax
from jax.experimental import pallas as pl
from jax.experimental.pallas import tpu as pltpu
from jax.experimental.pallas import tpu_sc as plsc

S = 2048
DIM = 1024
NUM_HEADS = 16
HEAD_DIM = DIM // NUM_HEADS
KN = 64
DEPTH = 8
SCALE = 1.0 / math.sqrt(HEAD_DIM)

QB = 128
NB = 512
WB = 384

NW = 32
RW = S // NW
CH = 8
NEG = -1e30


def _cantor_tables():
    pos = np.arange(S, dtype=np.float64)
    x = np.clip(pos / (S - 1), 1e-06, 1.0 - 1e-06)
    val = np.zeros(S)
    factor = 0.5
    for _ in range(DEPTH):
        x = x * 3.0
        digit = np.floor(x)
        x = x - digit
        val = val + np.where(digit == 2.0, factor, 0.0)
        factor *= 0.5
    c = val.astype(np.float32)
    order = np.argsort(c, kind="stable")
    rank = np.empty(S, np.int64)
    rank[order] = np.arange(S)
    dist = np.abs(c[:, None] - c[None, :])
    dkn = np.partition(dist, KN - 1, axis=1)[:, KN - 1]
    cand = dist <= dkn[:, None]
    rk = np.where(cand, rank[None, :], S).min(axis=1)
    lo = rk.astype(np.int64)
    hi = np.where(cand, rank[None, :], -1).max(axis=1).astype(np.int64)
    lo_s, hi_s = lo[order], hi[order]
    starts = []
    for b in range(S // QB):
        l = lo_s[b * QB:(b + 1) * QB].min()
        h = hi_s[b * QB:(b + 1) * QB].max()
        st = min((l // 8) * 8, S - WB)
        assert h - st + 1 <= WB
        starts.append(st)
    return (
        order.astype(np.int32),
        rank.astype(np.int32),
        np.asarray(starts, np.int32),
    )


_ORDER_NP, _RANK_NP, _STARTS_NP = _cantor_tables()
_STARTS16_NP = np.pad(_STARTS_NP, (0, 16 - S // QB))


def _bias_body(routes_hbm, rank_hbm, starts_hbm, bias_hbm,
               rts_v, rank_v, starts_v, buf_v, sem):
    wid = lax.axis_index("s") * 2 + lax.axis_index("c")
    base = wid * RW
    blk = base // QB
    pltpu.sync_copy(rank_hbm, rank_v)
    pltpu.sync_copy(starts_hbm, starts_v)
    pltpu.sync_copy(routes_hbm.at[pl.ds(base * KN, RW * KN)], rts_v)
    start_vec = plsc.load_gather(starts_v, [jnp.full((16,), blk, jnp.int32)])

    def fill(i, _):
        buf_v[pl.ds(i * 16, 16)] = jnp.full((16,), NEG, jnp.float32)
        return 0

    lax.fori_loop(0, RW * WB // 16, fill, 0)

    zeros16 = jnp.zeros((16,), jnp.float32)

    def row(r, _):
        for jj in range(KN // 16):
            rv = rts_v[pl.ds(r * KN + jj * 16, 16)]
            rk = plsc.load_gather(rank_v, [rv])
            loc = jnp.clip(rk - start_vec, 0, WB - 1) + r * WB
            plsc.store_scatter(buf_v, [loc], zeros16)
        return 0

    lax.fori_loop(0, RW, row, 0)
    pltpu.sync_copy(buf_v, bias_hbm.at[pl.ds(base * WB, RW * WB)])


def _build_bias(routes_sorted_flat):
    mesh = plsc.VectorSubcoreMesh(core_axis_name="c", subcore_axis_name="s")
    flat = pl.kernel(
        _bias_body,
        mesh=mesh,
        compiler_params=pltpu.CompilerParams(needs_layout_passes=False),
        out_type=jax.ShapeDtypeStruct((S * WB,), jnp.float32),
        scratch_types=[
            pltpu.VMEM((RW * KN,), jnp.int32),
            pltpu.VMEM((S,), jnp.int32),
            pltpu.VMEM((16,), jnp.int32),
            pltpu.VMEM((RW * WB,), jnp.float32),
            pltpu.SemaphoreType.DMA,
        ],
    )(routes_sorted_flat, jnp.asarray(_RANK_NP), jnp.asarray(_STARTS16_NP))
    return flat.reshape(S, WB)


def _qkv_body(x_ref, w_ref, b_ref, o_ref):
    o_ref[...] = (
        jnp.dot(x_ref[...], w_ref[...], preferred_element_type=jnp.float32)
        + b_ref[...]
    )


def _attn_body(starts_smem, bias_ref, qrow_ref, k_ref, v_ref, wout_ref, bout_ref, o_ref):
    i = pl.program_id(0)
    start = pl.multiple_of(starts_smem[i], 8)
    bias = bias_ref[...]
    outs = []
    for h in range(NUM_HEADS):
        q = qrow_ref[:, h * HEAD_DIM:(h + 1) * HEAD_DIM]
        k = k_ref[pl.ds(start, WB), h * HEAD_DIM:(h + 1) * HEAD_DIM]
        v = v_ref[pl.ds(start, WB), h * HEAD_DIM:(h + 1) * HEAD_DIM]
        scores = jax.lax.dot_general(
            q, k, (((1,), (1,)), ((), ())), preferred_element_type=jnp.float32
        ) * SCALE + bias
        m = jnp.max(scores, axis=1, keepdims=True)
        e = jnp.exp(scores - m)
        attn = e / jnp.sum(e, axis=1, keepdims=True)
        outs.append(jnp.dot(attn, v, preferred_element_type=jnp.float32))
    attn_out = jnp.concatenate(outs, axis=1)
    o_ref[...] = (
        jnp.dot(attn_out, wout_ref[...], preferred_element_type=jnp.float32)
        + bout_ref[...]
    )


@functools.partial(jax.jit, static_argnames=("interpret",))
def _run(x, W_qkv, b_qkv, W_out, b_out, routes, interpret=False):
    order = jnp.asarray(_ORDER_NP)
    x2 = jnp.take(x.reshape(S, DIM), order, axis=0)
    routes_sorted = jnp.take(routes, order, axis=0).reshape(S * KN)
    b_qkv2 = b_qkv.reshape(1, 3 * DIM)
    b_out2 = b_out.reshape(1, DIM)

    bias = _build_bias(routes_sorted)

    qkv = pl.pallas_call(
        _qkv_body,
        grid=(S // QB, (3 * DIM) // NB),
        in_specs=[
            pl.BlockSpec((QB, DIM), lambda i, j: (i, 0)),
            pl.BlockSpec((DIM, NB), lambda i, j: (0, j)),
            pl.BlockSpec((1, NB), lambda i, j: (0, j)),
        ],
        out_specs=pl.BlockSpec((QB, NB), lambda i, j: (i, j)),
        out_shape=jax.ShapeDtypeStruct((S, 3 * DIM), jnp.float32),
        interpret=interpret,
    )(x2, W_qkv, b_qkv2)

    out_sorted = pl.pallas_call(
        _attn_body,
        grid=(S // QB,),
        in_specs=[
            pl.BlockSpec(memory_space=pltpu.SMEM),
            pl.BlockSpec((QB, WB), lambda i: (i, 0)),
            pl.BlockSpec((QB, DIM), lambda i: (i, 0)),
            pl.BlockSpec((S, DIM), lambda i: (0, 1)),
            pl.BlockSpec((S, DIM), lambda i: (0, 2)),
            pl.BlockSpec((DIM, DIM), lambda i: (0, 0)),
            pl.BlockSpec((1, DIM), lambda i: (0, 0)),
        ],
        out_specs=pl.BlockSpec((QB, DIM), lambda i: (i, 0)),
        out_shape=jax.ShapeDtypeStruct((S, DIM), jnp.float32),
        interpret=interpret,
    )(jnp.asarray(_STARTS_NP), bias, qkv, qkv, qkv, W_out, b_out2)

    out = jnp.take(out_sorted, jnp.asarray(_RANK_NP), axis=0)
    return out.reshape(1, S, DIM)


def kernel(x, W_qkv, b_qkv, W_out, b_out, routes):
    return _run(x, W_qkv, b_qkv, W_out, b_out, routes)


# speedup vs baseline: 1.4912x; 1.4912x over previous
import functools
import math

import numpy as np

import jax
import jax.numpy as jnp
from jax import lax
from jax.experimental import pallas as pl
from jax.experimental.pallas import tpu as pltpu
from jax.experimental.pallas import tpu_sc as plsc

S = 2048
DIM = 1024
NUM_HEADS = 16
HEAD_DIM = DIM // NUM_HEADS
KN = 64
DEPTH = 8
SCALE = 1.0 / math.sqrt(HEAD_DIM)

QB = 256
NB = 512
WB = 384

NW = 32
RW = S // NW
CH = 8
NEG = -1e30


def _cantor_tables():
    pos = np.arange(S, dtype=np.float64)
    x = np.clip(pos / (S - 1), 1e-06, 1.0 - 1e-06)
    val = np.zeros(S)
    factor = 0.5
    for _ in range(DEPTH):
        x = x * 3.0
        digit = np.floor(x)
        x = x - digit
        val = val + np.where(digit == 2.0, factor, 0.0)
        factor *= 0.5
    c = val.astype(np.float32)
    order = np.argsort(c, kind="stable")
    rank = np.empty(S, np.int64)
    rank[order] = np.arange(S)
    dist = np.abs(c[:, None] - c[None, :])
    dkn = np.partition(dist, KN - 1, axis=1)[:, KN - 1]
    cand = dist <= dkn[:, None]
    rk = np.where(cand, rank[None, :], S).min(axis=1)
    lo = rk.astype(np.int64)
    hi = np.where(cand, rank[None, :], -1).max(axis=1).astype(np.int64)
    lo_s, hi_s = lo[order], hi[order]
    starts = []
    for b in range(S // QB):
        l = lo_s[b * QB:(b + 1) * QB].min()
        h = hi_s[b * QB:(b + 1) * QB].max()
        st = min((l // 8) * 8, S - WB)
        assert h - st + 1 <= WB
        starts.append(st)
    return (
        order.astype(np.int32),
        rank.astype(np.int32),
        np.asarray(starts, np.int32),
    )


_ORDER_NP, _RANK_NP, _STARTS_NP = _cantor_tables()
_STARTS16_NP = np.pad(_STARTS_NP, (0, 16 - S // QB))


def _bias_body(routes_hbm, rank_hbm, starts_hbm, bias_hbm,
               rts_v, rank_v, starts_v, buf_v, sem):
    wid = lax.axis_index("s") * 2 + lax.axis_index("c")
    base = wid * RW
    blk = base // QB
    pltpu.sync_copy(rank_hbm, rank_v)
    pltpu.sync_copy(starts_hbm, starts_v)
    start_vec = plsc.load_gather(starts_v, [jnp.full((16,), blk, jnp.int32)])

    def fill(i, _):
        buf_v[pl.ds(i * 16, 16)] = jnp.full((16,), NEG, jnp.float32)
        return 0

    lax.fori_loop(0, CH * WB // 16, fill, 0)

    zeros16 = jnp.zeros((16,), jnp.float32)
    negs16 = jnp.full((16,), NEG, jnp.float32)

    def chunk(cc, _):
        row0 = base + cc * CH
        pltpu.sync_copy(routes_hbm.at[pl.ds(row0 * KN, CH * KN)], rts_v)
        for r in range(CH):
            for jj in range(KN // 16):
                rv = rts_v[pl.ds(r * KN + jj * 16, 16)]
                rk = plsc.load_gather(rank_v, [rv])
                loc = jnp.clip(rk - start_vec, 0, WB - 1) + r * WB
                plsc.store_scatter(buf_v, [loc], zeros16)
        pltpu.sync_copy(buf_v, bias_hbm.at[pl.ds(row0 * WB, CH * WB)])
        for r in range(CH):
            for jj in range(KN // 16):
                rv = rts_v[pl.ds(r * KN + jj * 16, 16)]
                rk = plsc.load_gather(rank_v, [rv])
                loc = jnp.clip(rk - start_vec, 0, WB - 1) + r * WB
                plsc.store_scatter(buf_v, [loc], negs16)
        return 0

    lax.fori_loop(0, RW // CH, chunk, 0)


def _build_bias(routes_sorted_flat):
    mesh = plsc.VectorSubcoreMesh(core_axis_name="c", subcore_axis_name="s")
    flat = pl.kernel(
        _bias_body,
        mesh=mesh,
        compiler_params=pltpu.CompilerParams(needs_layout_passes=False),
        out_type=jax.ShapeDtypeStruct((S * WB,), jnp.float32),
        scratch_types=[
            pltpu.VMEM((CH * KN,), jnp.int32),
            pltpu.VMEM((S,), jnp.int32),
            pltpu.VMEM((16,), jnp.int32),
            pltpu.VMEM((CH * WB,), jnp.float32),
            pltpu.SemaphoreType.DMA,
        ],
    )(routes_sorted_flat, jnp.asarray(_RANK_NP), jnp.asarray(_STARTS16_NP))
    return flat.reshape(S, WB)


def _qkv_body(x_ref, w_ref, b_ref, o_ref):
    o_ref[...] = (
        jnp.dot(x_ref[...], w_ref[...], preferred_element_type=jnp.float32)
        + b_ref[...]
    )


def _attn_body(starts_smem, bias_ref, qrow_ref, k_ref, v_ref, wout_ref, bout_ref, o_ref):
    i = pl.program_id(0)
    start = pl.multiple_of(starts_smem[i], 8)
    bias = bias_ref[...]
    outs = []
    for h in range(NUM_HEADS):
        q = qrow_ref[:, h * HEAD_DIM:(h + 1) * HEAD_DIM]
        k = k_ref[pl.ds(start, WB), h * HEAD_DIM:(h + 1) * HEAD_DIM]
        v = v_ref[pl.ds(start, WB), h * HEAD_DIM:(h + 1) * HEAD_DIM]
        scores = jax.lax.dot_general(
            q, k, (((1,), (1,)), ((), ())), preferred_element_type=jnp.float32
        ) * SCALE + bias
        m = jnp.max(scores, axis=1, keepdims=True)
        e = jnp.exp(scores - m)
        attn = e / jnp.sum(e, axis=1, keepdims=True)
        outs.append(jnp.dot(attn, v, preferred_element_type=jnp.float32))
    attn_out = jnp.concatenate(outs, axis=1)
    o_ref[...] = (
        jnp.dot(attn_out, wout_ref[...], preferred_element_type=jnp.float32)
        + bout_ref[...]
    )


@functools.partial(jax.jit, static_argnames=("interpret",))
def _run(x, W_qkv, b_qkv, W_out, b_out, routes, interpret=False):
    order = jnp.asarray(_ORDER_NP)
    x2 = jnp.take(x.reshape(S, DIM), order, axis=0)
    routes_sorted = jnp.take(routes, order, axis=0).reshape(S * KN)
    b_qkv2 = b_qkv.reshape(1, 3 * DIM)
    b_out2 = b_out.reshape(1, DIM)

    bias = _build_bias(routes_sorted)

    qkv = pl.pallas_call(
        _qkv_body,
        grid=(S // QB, (3 * DIM) // NB),
        in_specs=[
            pl.BlockSpec((QB, DIM), lambda i, j: (i, 0)),
            pl.BlockSpec((DIM, NB), lambda i, j: (0, j)),
            pl.BlockSpec((1, NB), lambda i, j: (0, j)),
        ],
        out_specs=pl.BlockSpec((QB, NB), lambda i, j: (i, j)),
        out_shape=jax.ShapeDtypeStruct((S, 3 * DIM), jnp.float32),
        interpret=interpret,
    )(x2, W_qkv, b_qkv2)

    out_sorted = pl.pallas_call(
        _attn_body,
        grid=(S // QB,),
        in_specs=[
            pl.BlockSpec(memory_space=pltpu.SMEM),
            pl.BlockSpec((QB, WB), lambda i: (i, 0)),
            pl.BlockSpec((QB, DIM), lambda i: (i, 0)),
            pl.BlockSpec((S, DIM), lambda i: (0, 1)),
            pl.BlockSpec((S, DIM), lambda i: (0, 2)),
            pl.BlockSpec((DIM, DIM), lambda i: (0, 0)),
            pl.BlockSpec((1, DIM), lambda i: (0, 0)),
        ],
        out_specs=pl.BlockSpec((QB, DIM), lambda i: (i, 0)),
        out_shape=jax.ShapeDtypeStruct((S, DIM), jnp.float32),
        interpret=interpret,
    )(jnp.asarray(_STARTS_NP), bias, qkv, qkv, qkv, W_out, b_out2)

    out = jnp.take(out_sorted, jnp.asarray(_RANK_NP), axis=0)
    return out.reshape(1, S, DIM)


def kernel(x, W_qkv, b_qkv, W_out, b_out, routes):
    return _run(x, W_qkv, b_qkv, W_out, b_out, routes)
